# baseline (speedup 1.0000x reference)
import functools

import jax
import jax.numpy as jnp
from jax import lax
from jax.experimental import pallas as pl
from jax.experimental.pallas import tpu as pltpu
from jax.experimental.pallas import tpu_sc as plsc

_N = 10000
_D = 128
_E = 320000
_NC = 2
_NS = 16
_NW = _NC * _NS
_EPW = _E // _NW
_NP = 10240
_RPT = _NP // _NS
_EP = _NW * _NP
_EPWP = _NP
_CH = 128
_NCHUNK = _EPWP // _CH
_GRP = 8
_NGRP = _NCHUNK // _GRP
_FCH = 128
_GF = 10
_GS = 10
_FAST_C = 1
_NCHT = _EP // _FCH

_mesh = plsc.VectorSubcoreMesh(
    core_axis_name="c", subcore_axis_name="s", num_cores=_NC, num_subcores=_NS)


_CNT_KW = dict(
    out_type=jax.ShapeDtypeStruct((_NC * _NP, _D), jnp.float32),
    mesh=_mesh,
    scratch_types=[
        pltpu.VMEM((_GRP, _CH), jnp.int32),
        pltpu.VMEM((_CH, _D), jnp.float32),
        pltpu.VMEM_SHARED((_NP, _D), jnp.float32),
        pltpu.SemaphoreType.DMA,
    ],
)


def _sc_count_body(dst_hbm, zero_hbm, ones_hbm, cnt_hbm,
                   dst_v, ones_v, cnt_sp, sem):
  c = lax.axis_index("c")
  s = lax.axis_index("s")
  w = c * _NS + s
  r0 = s * _RPT
  pltpu.sync_copy(zero_hbm.at[pl.ds(r0, _RPT)], cnt_sp.at[pl.ds(r0, _RPT)])
  pltpu.sync_copy(ones_hbm, ones_v)
  plsc.subcore_barrier()

  def group(g, carry):
    pltpu.sync_copy(dst_hbm.at[w, pl.ds(g * _GRP, _GRP)], dst_v)
    for j in range(_GRP):
      pltpu.async_copy(ones_v, cnt_sp.at[dst_v.at[j]], sem, add=True)
    for j in range(_GRP):
      pltpu.make_async_copy(ones_v, cnt_sp.at[dst_v.at[j]], sem).wait()
    return carry
  lax.fori_loop(0, _NGRP, group, 0)
  plsc.subcore_barrier()
  o0 = c * _NP + r0
  pltpu.sync_copy(cnt_sp.at[pl.ds(r0, _RPT)], cnt_hbm.at[pl.ds(o0, _RPT)])


_SEG_KW = dict(
    out_type=jax.ShapeDtypeStruct((_NC * _NP, _D), jnp.float32),
    mesh=_mesh,
    scratch_types=[
        pltpu.VMEM((2 * _GRP, _FCH), jnp.int32),
        pltpu.VMEM((2 * _GRP, _FCH), jnp.int32),
        pltpu.VMEM((_FCH, _D), jnp.float32),
        pltpu.VMEM((_FCH, _D), jnp.float32),
        pltpu.VMEM_SHARED((_NP, _D), jnp.float32),
        pltpu.SemaphoreType.DMA,
        pltpu.SemaphoreType.DMA,
        pltpu.SemaphoreType.DMA,
        pltpu.SemaphoreType.DMA,
    ],
)


def _sc_segsum_body(x_hbm, src_hbm, dst_hbm, zero_hbm, out_hbm,
                    src_v, dst_v, r0b, r1b, acc_sp, g0, g1, s0, s1):
  rows = (r0b, r1b)
  gsem = (g0, g1)
  ssem = (s0, s1)
  c = lax.axis_index("c")
  s = lax.axis_index("s")
  r0 = s * _RPT
  ngrp = jnp.where(c == _FAST_C, _GF, _GS)
  grp0 = jnp.where(c == _FAST_C, s * _GF, _NS * _GF + s * _GS)

  pltpu.sync_copy(zero_hbm.at[pl.ds(r0, _RPT)], acc_sp.at[pl.ds(r0, _RPT)])
  pltpu.sync_copy(src_hbm.at[pl.ds(grp0 * _GRP, _GRP)], src_v.at[pl.ds(0, _GRP)])
  pltpu.sync_copy(dst_hbm.at[pl.ds(grp0 * _GRP, _GRP)], dst_v.at[pl.ds(0, _GRP)])
  plsc.subcore_barrier()
  pltpu.async_copy(x_hbm.at[src_v.at[0]], rows[0], gsem[0])

  def group(g, carry2):
    half_row = (g % 2) * _GRP
    nhalf = ((g + 1) % 2) * _GRP

    def chunk(j, row_j, row_j1):
      b = j % 2
      b1 = (j + 1) % 2
      pltpu.make_async_copy(x_hbm.at[src_v.at[row_j]], rows[b],
                            gsem[b]).wait()
      pltpu.async_copy(rows[b], acc_sp.at[dst_v.at[row_j]], ssem[b],
                       add=True)
      def drain():
        pltpu.make_async_copy(rows[b1], acc_sp.at[dst_v.at[row_j]],
                              ssem[b1]).wait()
      if j == 0:
        pl.when(g > 0)(drain)
      else:
        drain()
      if j < _GRP - 1:
        pltpu.async_copy(x_hbm.at[src_v.at[row_j1]], rows[b1], gsem[b1])
      else:
        @pl.when(g + 1 < ngrp)
        def _():
          pltpu.async_copy(x_hbm.at[src_v.at[row_j1]], rows[b1], gsem[b1])

    chunk(0, half_row + 0, half_row + 1)

    @pl.when(g + 1 < ngrp)
    def _stage_next():
      pltpu.sync_copy(src_hbm.at[pl.ds((grp0 + g + 1) * _GRP, _GRP)],
                      src_v.at[pl.ds(nhalf, _GRP)])
      pltpu.sync_copy(dst_hbm.at[pl.ds((grp0 + g + 1) * _GRP, _GRP)],
                      dst_v.at[pl.ds(nhalf, _GRP)])

    for j in range(1, _GRP):
      row_j1 = half_row + j + 1 if j < _GRP - 1 else nhalf
      chunk(j, half_row + j, row_j1)
    return carry2

  lax.fori_loop(0, ngrp, group, 0)
  pltpu.make_async_copy(rows[1], acc_sp.at[dst_v.at[0]], ssem[1]).wait()
  plsc.subcore_barrier()
  o0 = c * _NP + r0
  pltpu.sync_copy(acc_sp.at[pl.ds(r0, _RPT)], out_hbm.at[pl.ds(o0, _RPT)])


_sc_count = pl.kernel(_sc_count_body, **_CNT_KW)
_sc_segsum = pl.kernel(_sc_segsum_body, **_SEG_KW)


_BN = 1000


def _tc_body(relu, p0_r, p1_r, c0_r, c1_r, x_r, wl_r, wr_r, bl_r, o_r):
  cnt = c0_r[:, 0:1] + c1_r[:, 0:1]
  agg = (p0_r[...] + p1_r[...]) / jnp.maximum(cnt, 1.0)
  y = (jnp.dot(agg, wl_r[...], preferred_element_type=jnp.float32)
       + jnp.dot(x_r[...], wr_r[...], preferred_element_type=jnp.float32)
       + bl_r[...])
  if relu:
    y = jnp.maximum(y, 0.0)
  o_r[...] = y


def _make_tc_layer(relu):
  row_spec = pl.BlockSpec((_BN, _D), lambda i: (i, 0))
  cnt_spec = pl.BlockSpec((_BN, 16), lambda i: (i, 0))
  w_spec = pl.BlockSpec((_D, _D), lambda i: (0, 0))
  b_spec = pl.BlockSpec((1, _D), lambda i: (0, 0))
  return pl.pallas_call(
      functools.partial(_tc_body, relu),
      grid=(_N // _BN,),
      in_specs=[row_spec, row_spec, cnt_spec, cnt_spec, row_spec, w_spec,
                w_spec, b_spec],
      out_specs=row_spec,
      out_shape=jax.ShapeDtypeStruct((_N, _D), jnp.float32),
  )


_tc_layer_relu = _make_tc_layer(True)
_tc_layer_lin = _make_tc_layer(False)


def kernel(x, edge_index, Wl1, bl1, Wr1, Wl2, bl2, Wr2):
  npad = _EP - _E
  pad_dst = _N + jnp.arange(npad, dtype=jnp.int32) % (_NP - _N)
  src = jnp.concatenate(
      [edge_index[0].astype(jnp.int32), jnp.zeros((npad,), jnp.int32)])
  dst = jnp.concatenate([edge_index[1].astype(jnp.int32), pad_dst])
  srcf = src.reshape(_NCHT, _FCH)
  dstf = dst.reshape(_NCHT, _FCH)
  dst = dst.reshape(_NW, _NCHUNK, _CH)
  zero_nd = jnp.zeros((_NP, _D), jnp.float32)

  ones_rows = jnp.ones((_CH, _D), jnp.float32)
  part1 = _sc_segsum(x, srcf, dstf, zero_nd)
  cntp = _sc_count(dst, zero_nd, ones_rows)
  cnts = cntp[:, :16]
  h = _tc_layer_relu(part1[:_N], part1[_NP:_NP + _N], cnts[:_N],
                     cnts[_NP:_NP + _N], x, Wl1, Wr1, bl1.reshape(1, _D))
  part2 = _sc_segsum(h, srcf, dstf, zero_nd)
  out = _tc_layer_lin(part2[:_N], part2[_NP:_NP + _N], cnts[:_N],
                      cnts[_NP:_NP + _N], h, Wl2, Wr2, bl2.reshape(1, _D))
  return out


# speedup vs baseline: 1.0762x; 1.0762x over previous
import functools

import jax
import jax.numpy as jnp
from jax import lax
from jax.experimental import pallas as pl
from jax.experimental.pallas import tpu as pltpu
from jax.experimental.pallas import tpu_sc as plsc

_N = 10000
_D = 128
_E = 320000
_NC = 2
_NS = 16
_NW = _NC * _NS
_EPW = _E // _NW
_NP = 10240
_RPT = _NP // _NS
_EP = _NW * _NP
_EPWP = _NP
_CH = 128
_NCHUNK = _EPWP // _CH
_GRP = 8
_NGRP = _NCHUNK // _GRP
_FCH = 128
_GF = 15
_GS = 5
_FAST_C = 1
_NCHT = _EP // _FCH

_mesh = plsc.VectorSubcoreMesh(
    core_axis_name="c", subcore_axis_name="s", num_cores=_NC, num_subcores=_NS)


_CNT_KW = dict(
    out_type=jax.ShapeDtypeStruct((_NC * _NP, _D), jnp.float32),
    mesh=_mesh,
    scratch_types=[
        pltpu.VMEM((_GRP, _CH), jnp.int32),
        pltpu.VMEM((_CH, _D), jnp.float32),
        pltpu.VMEM_SHARED((_NP, _D), jnp.float32),
        pltpu.SemaphoreType.DMA,
    ],
)


def _sc_count_body(dst_hbm, zero_hbm, ones_hbm, cnt_hbm,
                   dst_v, ones_v, cnt_sp, sem):
  c = lax.axis_index("c")
  s = lax.axis_index("s")
  w = c * _NS + s
  r0 = s * _RPT
  pltpu.sync_copy(zero_hbm.at[pl.ds(r0, _RPT)], cnt_sp.at[pl.ds(r0, _RPT)])
  pltpu.sync_copy(ones_hbm, ones_v)
  plsc.subcore_barrier()

  def group(g, carry):
    pltpu.sync_copy(dst_hbm.at[w, pl.ds(g * _GRP, _GRP)], dst_v)
    for j in range(_GRP):
      pltpu.async_copy(ones_v, cnt_sp.at[dst_v.at[j]], sem, add=True)
    for j in range(_GRP):
      pltpu.make_async_copy(ones_v, cnt_sp.at[dst_v.at[j]], sem).wait()
    return carry
  lax.fori_loop(0, _NGRP, group, 0)
  plsc.subcore_barrier()
  o0 = c * _NP + r0
  pltpu.sync_copy(cnt_sp.at[pl.ds(r0, _RPT)], cnt_hbm.at[pl.ds(o0, _RPT)])


_SEG_KW = dict(
    out_type=jax.ShapeDtypeStruct((_NC * _NP, _D), jnp.float32),
    mesh=_mesh,
    scratch_types=[
        pltpu.VMEM((2 * _GRP, _FCH), jnp.int32),
        pltpu.VMEM((2 * _GRP, _FCH), jnp.int32),
        pltpu.VMEM((_FCH, _D), jnp.float32),
        pltpu.VMEM((_FCH, _D), jnp.float32),
        pltpu.VMEM_SHARED((_NP, _D), jnp.float32),
        pltpu.SemaphoreType.DMA,
        pltpu.SemaphoreType.DMA,
        pltpu.SemaphoreType.DMA,
        pltpu.SemaphoreType.DMA,
    ],
)


def _sc_segsum_body(x_hbm, src_hbm, dst_hbm, zero_hbm, out_hbm,
                    src_v, dst_v, r0b, r1b, acc_sp, g0, g1, s0, s1):
  rows = (r0b, r1b)
  gsem = (g0, g1)
  ssem = (s0, s1)
  c = lax.axis_index("c")
  s = lax.axis_index("s")
  r0 = s * _RPT
  ngrp = jnp.where(c == _FAST_C, _GF, _GS)
  grp0 = jnp.where(c == _FAST_C, s * _GF, _NS * _GF + s * _GS)

  pltpu.sync_copy(zero_hbm.at[pl.ds(r0, _RPT)], acc_sp.at[pl.ds(r0, _RPT)])
  pltpu.sync_copy(src_hbm.at[pl.ds(grp0 * _GRP, _GRP)], src_v.at[pl.ds(0, _GRP)])
  pltpu.sync_copy(dst_hbm.at[pl.ds(grp0 * _GRP, _GRP)], dst_v.at[pl.ds(0, _GRP)])
  plsc.subcore_barrier()
  pltpu.async_copy(x_hbm.at[src_v.at[0]], rows[0], gsem[0])

  def group(g, carry2):
    half_row = (g % 2) * _GRP
    nhalf = ((g + 1) % 2) * _GRP

    def chunk(j, row_j, row_j1):
      b = j % 2
      b1 = (j + 1) % 2
      pltpu.make_async_copy(x_hbm.at[src_v.at[row_j]], rows[b],
                            gsem[b]).wait()
      pltpu.async_copy(rows[b], acc_sp.at[dst_v.at[row_j]], ssem[b],
                       add=True)
      def drain():
        pltpu.make_async_copy(rows[b1], acc_sp.at[dst_v.at[row_j]],
                              ssem[b1]).wait()
      if j == 0:
        pl.when(g > 0)(drain)
      else:
        drain()
      if j < _GRP - 1:
        pltpu.async_copy(x_hbm.at[src_v.at[row_j1]], rows[b1], gsem[b1])
      else:
        @pl.when(g + 1 < ngrp)
        def _():
          pltpu.async_copy(x_hbm.at[src_v.at[row_j1]], rows[b1], gsem[b1])

    chunk(0, half_row + 0, half_row + 1)

    @pl.when(g + 1 < ngrp)
    def _stage_next():
      pltpu.sync_copy(src_hbm.at[pl.ds((grp0 + g + 1) * _GRP, _GRP)],
                      src_v.at[pl.ds(nhalf, _GRP)])
      pltpu.sync_copy(dst_hbm.at[pl.ds((grp0 + g + 1) * _GRP, _GRP)],
                      dst_v.at[pl.ds(nhalf, _GRP)])

    for j in range(1, _GRP):
      row_j1 = half_row + j + 1 if j < _GRP - 1 else nhalf
      chunk(j, half_row + j, row_j1)
    return carry2

  lax.fori_loop(0, ngrp, group, 0)
  pltpu.make_async_copy(rows[1], acc_sp.at[dst_v.at[0]], ssem[1]).wait()
  plsc.subcore_barrier()
  o0 = c * _NP + r0
  pltpu.sync_copy(acc_sp.at[pl.ds(r0, _RPT)], out_hbm.at[pl.ds(o0, _RPT)])


_sc_count = pl.kernel(_sc_count_body, **_CNT_KW)
_sc_segsum = pl.kernel(_sc_segsum_body, **_SEG_KW)


_BN = 1000


def _tc_body(relu, p0_r, p1_r, c0_r, c1_r, x_r, wl_r, wr_r, bl_r, o_r):
  cnt = c0_r[:, 0:1] + c1_r[:, 0:1]
  agg = (p0_r[...] + p1_r[...]) / jnp.maximum(cnt, 1.0)
  y = (jnp.dot(agg, wl_r[...], preferred_element_type=jnp.float32)
       + jnp.dot(x_r[...], wr_r[...], preferred_element_type=jnp.float32)
       + bl_r[...])
  if relu:
    y = jnp.maximum(y, 0.0)
  o_r[...] = y


def _make_tc_layer(relu):
  row_spec = pl.BlockSpec((_BN, _D), lambda i: (i, 0))
  cnt_spec = pl.BlockSpec((_BN, 16), lambda i: (i, 0))
  w_spec = pl.BlockSpec((_D, _D), lambda i: (0, 0))
  b_spec = pl.BlockSpec((1, _D), lambda i: (0, 0))
  return pl.pallas_call(
      functools.partial(_tc_body, relu),
      grid=(_N // _BN,),
      in_specs=[row_spec, row_spec, cnt_spec, cnt_spec, row_spec, w_spec,
                w_spec, b_spec],
      out_specs=row_spec,
      out_shape=jax.ShapeDtypeStruct((_N, _D), jnp.float32),
  )


_tc_layer_relu = _make_tc_layer(True)
_tc_layer_lin = _make_tc_layer(False)


def kernel(x, edge_index, Wl1, bl1, Wr1, Wl2, bl2, Wr2):
  npad = _EP - _E
  pad_dst = _N + jnp.arange(npad, dtype=jnp.int32) % (_NP - _N)
  src = jnp.concatenate(
      [edge_index[0].astype(jnp.int32), jnp.zeros((npad,), jnp.int32)])
  dst = jnp.concatenate([edge_index[1].astype(jnp.int32), pad_dst])
  srcf = src.reshape(_NCHT, _FCH)
  dstf = dst.reshape(_NCHT, _FCH)
  dst = dst.reshape(_NW, _NCHUNK, _CH)
  zero_nd = jnp.zeros((_NP, _D), jnp.float32)

  ones_rows = jnp.ones((_CH, _D), jnp.float32)
  part1 = _sc_segsum(x, srcf, dstf, zero_nd)
  cntp = _sc_count(dst, zero_nd, ones_rows)
  cnts = cntp[:, :16]
  h = _tc_layer_relu(part1[:_N], part1[_NP:_NP + _N], cnts[:_N],
                     cnts[_NP:_NP + _N], x, Wl1, Wr1, bl1.reshape(1, _D))
  part2 = _sc_segsum(h, srcf, dstf, zero_nd)
  out = _tc_layer_lin(part2[:_N], part2[_NP:_NP + _N], cnts[:_N],
                      cnts[_NP:_NP + _N], h, Wl2, Wr2, bl2.reshape(1, _D))
  return out
